# Initial kernel scaffold; baseline (speedup 1.0000x reference)
import jax
import jax.numpy as jnp
from jax.experimental import pallas as pl


def kernel(feat, edge_index, mask_candidate_set, W0, b0, Wg1, bg1, Wg2, bg2, Wg3, bg3, Ws1, bs1, Ws2, bs2, Wt1, bt1, Wt2, bt2):
    raise NotImplementedError("write your pallas kernel here")


# speedup vs baseline: 16.9371x; 16.9371x over previous
import functools

import jax
import jax.numpy as jnp
from jax import lax
from jax.experimental import pallas as pl
from jax.experimental.pallas import tpu as pltpu
from jax.experimental.pallas import tpu_sc as plsc

N = 10000
NP = 10112
SLICE = NP // 16
CAND = 7
MAXN = 9993
E = 160000
CHUNK = 128
EP = 163840
NROWS = EP // CHUNK
RPT = NROWS // 32
DUMP = N

@functools.lru_cache(maxsize=None)
def _mesh():
    return plsc.VectorSubcoreMesh(core_axis_name="c", subcore_axis_name="s")


def _relu6(v):
    return jnp.clip(v, 0.0, 6.0)


def _mm(a, b):
    return jax.lax.dot_general(a, b, (((1,), (0,)), ((), ())),
                               preferred_element_type=jnp.float32)


@functools.lru_cache(maxsize=None)
def _make_deg():
    @functools.partial(
        pl.kernel,
        out_type=jax.ShapeDtypeStruct((2, NP, 16), jnp.float32),
        mesh=_mesh(),
        scratch_types=[
            pltpu.VMEM((RPT, CHUNK), jnp.int32),
            pltpu.VMEM((CHUNK, 16), jnp.float32),
            pltpu.VMEM_SHARED((NP, 16), jnp.float32),
        ],
        compiler_params=pltpu.CompilerParams(use_tc_tiling_on_sc=False),
    )
    def deg_kernel(dstr_hbm, ones_hbm, zeros_hbm, out_hbm, didx, ones_v, acc):
        c = lax.axis_index("c")
        s = lax.axis_index("s")
        tile = s * 2 + c
        pltpu.sync_copy(zeros_hbm.at[pl.ds(s * SLICE, SLICE)],
                        acc.at[pl.ds(s * SLICE, SLICE)])
        pltpu.sync_copy(dstr_hbm.at[pl.ds(tile * RPT, RPT)], didx)
        pltpu.sync_copy(ones_hbm, ones_v)
        plsc.subcore_barrier()
        for g in range(RPT):
            pltpu.sync_copy(ones_v, acc.at[didx.at[g]], add=True)
        plsc.subcore_barrier()
        pltpu.sync_copy(acc.at[pl.ds(s * SLICE, SLICE)],
                        out_hbm.at[c, pl.ds(s * SLICE, SLICE)])

    return deg_kernel


@functools.lru_cache(maxsize=None)
def _make_segsum(F):
    @functools.partial(
        pl.kernel,
        out_type=jax.ShapeDtypeStruct((2, NP, F), jnp.float32),
        mesh=_mesh(),
        scratch_types=[
            pltpu.VMEM((RPT, CHUNK), jnp.int32),
            pltpu.VMEM((RPT, CHUNK), jnp.int32),
            pltpu.VMEM((CHUNK, F), jnp.float32),
            pltpu.VMEM((CHUNK, F), jnp.float32),
            pltpu.VMEM_SHARED((NP, F), jnp.float32),
            pltpu.SemaphoreType.DMA,
            pltpu.SemaphoreType.DMA,
        ],
        compiler_params=pltpu.CompilerParams(use_tc_tiling_on_sc=False),
    )
    def seg_kernel(y_hbm, srcr_hbm, dstr_hbm, zeros_hbm, out_hbm,
                   sidx, didx, rows_a, rows_b, acc, sem_a, sem_b):
        c = lax.axis_index("c")
        s = lax.axis_index("s")
        tile = s * 2 + c

        pltpu.sync_copy(zeros_hbm.at[pl.ds(s * SLICE, SLICE)],
                        acc.at[pl.ds(s * SLICE, SLICE)])
        pltpu.sync_copy(srcr_hbm.at[pl.ds(tile * RPT, RPT)], sidx)
        pltpu.sync_copy(dstr_hbm.at[pl.ds(tile * RPT, RPT)], didx)
        plsc.subcore_barrier()
        bufs = (rows_a, rows_b)
        sems = (sem_a, sem_b)
        h = pltpu.async_copy(y_hbm.at[sidx.at[0]], bufs[0], sems[0])
        for g in range(RPT):
            cur = bufs[g % 2]
            h_next = None
            if g + 1 < RPT:
                h_next = pltpu.async_copy(y_hbm.at[sidx.at[g + 1]],
                                          bufs[(g + 1) % 2],
                                          sems[(g + 1) % 2])
            h.wait()
            pltpu.sync_copy(cur, acc.at[didx.at[g]], add=True)
            h = h_next
        plsc.subcore_barrier()
        pltpu.sync_copy(acc.at[pl.ds(s * SLICE, SLICE)],
                        out_hbm.at[c, pl.ds(s * SLICE, SLICE)])

    return seg_kernel


def _deg_sc(*a):
    return _make_deg()(*a)


def _seg16(*a):
    return _make_segsum(16)(*a)


def _seg24(*a):
    return _make_segsum(24)(*a)


def _seg32(*a):
    return _make_segsum(32)(*a)


def _tc1_body(feat_ref, d0_ref, d1_ref, w0_ref, b0_ref, w1_ref,
              xw_ref, y_ref, dinv_ref):
    deg = d0_ref[...] + d1_ref[...] + 1.0
    dinv = lax.rsqrt(deg)
    x0 = _relu6(_mm(feat_ref[...], w0_ref[...]) + b0_ref[...])
    xw = _mm(x0, w1_ref[...])
    xw_ref[...] = xw
    y_ref[...] = xw * dinv
    dinv_ref[...] = dinv


def _tc1(featp, deg0, deg1, W0, b0, Wg1):
    return pl.pallas_call(
        _tc1_body,
        out_shape=[
            jax.ShapeDtypeStruct((NP, 16), jnp.float32),
            jax.ShapeDtypeStruct((NP, 16), jnp.float32),
            jax.ShapeDtypeStruct((NP, 1), jnp.float32),
        ],
    )(featp, deg0, deg1, W0, b0, Wg1)


def _tc_layer_body(z0_ref, z1_ref, xw_ref, dinv_ref, b_ref, wn_ref,
                   xwn_ref, yn_ref):
    dinv = dinv_ref[...]
    x = _relu6((z0_ref[...] + z1_ref[...] + xw_ref[...] * dinv) * dinv
               + b_ref[...])
    xwn = _mm(x, wn_ref[...])
    xwn_ref[...] = xwn
    yn_ref[...] = xwn * dinv


def _tc_layer(z, xw, dinv, b, Wn, Fn):
    return pl.pallas_call(
        _tc_layer_body,
        out_shape=[
            jax.ShapeDtypeStruct((NP, Fn), jnp.float32),
            jax.ShapeDtypeStruct((NP, Fn), jnp.float32),
        ],
    )(z[0], z[1], xw, dinv, b, Wn)


def _head_body(z0_ref, z1_ref, xw_ref, dinv_ref, bg_ref, mask_ref,
               ws1_ref, bs1_ref, ws2_ref, bs2_ref,
               wt1_ref, bt1_ref, wt2_ref, bt2_ref,
               sp_ref, bsrc_ref, tp_ref, btgt_ref):
    dinv = dinv_ref[...]
    x = _relu6((z0_ref[...] + z1_ref[...] + xw_ref[...] * dinv) * dinv
               + bg_ref[...])
    row = lax.broadcasted_iota(jnp.int32, (NP, 1), 0)
    valid = row < N

    h = _relu6(_mm(x, ws1_ref[...]) + bs1_ref[...])
    sl = _mm(h, ws2_ref[...]) + bs2_ref[...]
    slv = jnp.where(valid, sl, -1e30)
    e = jnp.where(valid, jnp.exp(sl - jnp.max(slv)), 0.0)
    sp = e / jnp.sum(e)
    m = mask_ref[...] > 0.0
    sp_ref[...] = jnp.where(m, 0.0, sp)
    am = jnp.where(valid, jnp.where(m, -1.0, sp), -2.0)
    best = jnp.min(jnp.where(am == jnp.max(am), row, NP))
    bsrc_ref[...] = jnp.full((1, 1), best, jnp.int32)

    xs = jnp.sum(jnp.where(row == best, x, 0.0), axis=0, keepdims=True)
    wt1 = wt1_ref[...]
    th = _relu6(_mm(x, wt1[0:32, :]) + _mm(xs, wt1[32:64, :]) + bt1_ref[...])
    tl = _mm(th, wt2_ref[...]) + bt2_ref[...]
    tlv = jnp.where(valid, tl, -1e30)
    te = jnp.where(valid, jnp.exp(tl - jnp.max(tlv)), 0.0)
    tp = te / jnp.sum(te)
    tvalid = row < MAXN
    tp_ref[...] = jnp.where(tvalid, tp, 0.0)
    am2 = jnp.where(tvalid, tp, -1.0)
    best2 = jnp.min(jnp.where(am2 == jnp.max(am2), row, NP))
    btgt_ref[...] = jnp.full((1, 1), best2, jnp.int32)


def _head(z, xw, dinv, bg, maskp, Ws1, bs1, Ws2, bs2, Wt1, bt1, Wt2, bt2):
    return pl.pallas_call(
        _head_body,
        out_shape=[
            jax.ShapeDtypeStruct((NP, 1), jnp.float32),
            jax.ShapeDtypeStruct((1, 1), jnp.int32),
            jax.ShapeDtypeStruct((NP, 1), jnp.float32),
            jax.ShapeDtypeStruct((1, 1), jnp.int32),
        ],
    )(z[0], z[1], xw, dinv, bg, maskp,
      Ws1, bs1, Ws2, bs2, Wt1, bt1, Wt2, bt2)


def kernel(feat, edge_index, mask_candidate_set,
           W0, b0, Wg1, bg1, Wg2, bg2, Wg3, bg3,
           Ws1, bs1, Ws2, bs2, Wt1, bt1, Wt2, bt2):
    src = edge_index[0].astype(jnp.int32)
    dst = edge_index[1].astype(jnp.int32)
    pad = jnp.full((EP - E,), DUMP, jnp.int32)
    srcr = jnp.concatenate([src, pad]).reshape(NROWS, CHUNK)
    dstr = jnp.concatenate([dst, pad]).reshape(NROWS, CHUNK)

    featp = jnp.pad(feat, ((0, NP - N), (0, 0)))
    maskp = jnp.pad(mask_candidate_set.astype(jnp.float32),
                    (0, NP - N))[:, None]
    ones_c = jnp.ones((CHUNK, 16), jnp.float32)
    z1_deg = jnp.zeros((NP, 16), jnp.float32)
    z16 = jnp.zeros((NP, 16), jnp.float32)
    z24 = jnp.zeros((NP, 24), jnp.float32)
    z32 = jnp.zeros((NP, 32), jnp.float32)

    degs = _deg_sc(dstr, ones_c, z1_deg)
    deg0 = degs[0][:, :1]
    deg1 = degs[1][:, :1]

    xw1, y1, dinv = _tc1(featp, deg0, deg1, W0, b0, Wg1)
    z1 = _seg16(y1, srcr, dstr, z16)
    xw2, y2 = _tc_layer(z1, xw1, dinv, bg1, Wg2, 24)
    z2 = _seg24(y2, srcr, dstr, z24)
    xw3, y3 = _tc_layer(z2, xw2, dinv, bg2, Wg3, 32)
    z3 = _seg32(y3, srcr, dstr, z32)

    sp, bsrc, tp, btgt = _head(z3, xw3, dinv, bg3, maskp,
                               Ws1, bs1, Ws2, bs2, Wt1, bt1, Wt2, bt2)
    return (sp[:N], bsrc.reshape(()), tp[:N], btgt.reshape(()))
